# initial kernel scaffold (unmeasured)
import jax
import jax.numpy as jnp
from jax import lax
from jax.experimental import pallas as pl
from jax.experimental.pallas import tpu as pltpu

N_DEV = 4


def kernel(x, w_mat, scale_x, scale_w):
    m, _ = x.shape
    n = w_mat.shape[1]
    chunk = m // N_DEV

    xb = x.astype(jnp.bfloat16)
    wb = w_mat.astype(jnp.bfloat16)

    def body(x_ref, w_ref, sx_ref, sw_ref, out_ref,
             acc, send_buf, rs_recv, ag_buf,
             rs_send_sem, rs_recv_sem, ag_send_sem, ag_recv_sem, copy_sem):
        me = lax.axis_index("i")
        left = lax.rem(me + N_DEV - 1, N_DEV)
        right = lax.rem(me + 1, N_DEV)
        scale = sx_ref[0] * sw_ref[0]

        barrier = pltpu.get_barrier_semaphore()
        for nbr in (left, right):
            pl.semaphore_signal(barrier, inc=1, device_id=(nbr,),
                                device_id_type=pl.DeviceIdType.MESH)
        pl.semaphore_wait(barrier, 2)

        def partial(c):
            xs = x_ref[pl.ds(c * chunk, chunk), :]
            return lax.dot_general(xs, w_ref[...], (((1,), (0,)), ((), ())),
                                   preferred_element_type=jnp.float32)

        for h in range(N_DEV - 1):
            s_chunk = lax.rem(me + N_DEV - h, N_DEV)
            p = partial(s_chunk)
            if h > 0:
                p = p + rs_recv[h - 1].astype(jnp.float32)
            send_buf[h % 2] = p.astype(jnp.bfloat16)
            rdma = pltpu.make_async_remote_copy(
                src_ref=send_buf.at[h % 2],
                dst_ref=rs_recv.at[h],
                send_sem=rs_send_sem.at[h],
                recv_sem=rs_recv_sem.at[h],
                device_id=(right,),
                device_id_type=pl.DeviceIdType.MESH,
            )
            rdma.start()
            rdma.wait()

        own = lax.rem(me + 1, N_DEV)
        y = (partial(own) + rs_recv[N_DEV - 2].astype(jnp.float32)) * scale
        z = y * (1.0 / (1.0 + jnp.exp(-jnp.clip(y, -60.0, 60.0))))
        acc[...] = z
        for c in range(N_DEV):
            @pl.when(own == c)
            def _():
                ag_buf[c] = acc[...].astype(jnp.bfloat16)
        cp = pltpu.make_async_copy(
            acc, out_ref.at[pl.ds(own * chunk, chunk), :], copy_sem)
        cp.start()
        cp.wait()

        for a in range(N_DEV - 1):
            c_send = lax.rem(own + N_DEV - a, N_DEV)
            c_recv = lax.rem(own + N_DEV - a - 1, N_DEV)
            rdma = pltpu.make_async_remote_copy(
                src_ref=ag_buf.at[c_send],
                dst_ref=ag_buf.at[c_send],
                send_sem=ag_send_sem.at[a],
                recv_sem=ag_recv_sem.at[a],
                device_id=(right,),
                device_id_type=pl.DeviceIdType.MESH,
            )
            rdma.start()
            rdma.wait()
            for c in range(N_DEV):
                @pl.when(c_recv == c)
                def _():
                    acc[...] = ag_buf[c].astype(jnp.float32)
            cp = pltpu.make_async_copy(
                acc, out_ref.at[pl.ds(c_recv * chunk, chunk), :], copy_sem)
            cp.start()
            cp.wait()

    return pl.pallas_call(
        body,
        out_shape=jax.ShapeDtypeStruct((m, n), jnp.float32),
        in_specs=[
            pl.BlockSpec(memory_space=pltpu.VMEM),
            pl.BlockSpec(memory_space=pltpu.VMEM),
            pl.BlockSpec(memory_space=pltpu.SMEM),
            pl.BlockSpec(memory_space=pltpu.SMEM),
        ],
        out_specs=pl.BlockSpec(memory_space=pltpu.ANY),
        scratch_shapes=[
            pltpu.VMEM((chunk, n), jnp.float32),
            pltpu.VMEM((2, chunk, n), jnp.bfloat16),
            pltpu.VMEM((N_DEV - 1, chunk, n), jnp.bfloat16),
            pltpu.VMEM((N_DEV, chunk, n), jnp.bfloat16),
            pltpu.SemaphoreType.DMA((N_DEV - 1,)),
            pltpu.SemaphoreType.DMA((N_DEV - 1,)),
            pltpu.SemaphoreType.DMA((N_DEV - 1,)),
            pltpu.SemaphoreType.DMA((N_DEV - 1,)),
            pltpu.SemaphoreType.DMA,
        ],
        compiler_params=pltpu.CompilerParams(collective_id=0),
    )(xb, wb, scale_x, scale_w)


# baseline (device time: 361028 ns/iter reference)
import jax
import jax.numpy as jnp
from jax import lax
from jax.experimental import pallas as pl
from jax.experimental.pallas import tpu as pltpu

N_DEV = 4


def kernel(x, w_mat, scale_x, scale_w):
    m, _ = x.shape
    n = w_mat.shape[1]
    chunk = m // N_DEV

    xb = x.astype(jnp.bfloat16)
    wb = w_mat.astype(jnp.bfloat16)

    def body(x_ref, w_ref, sx_ref, sw_ref, out_ref,
             acc, send_buf, rs_recv, ag_buf,
             rs_send_sem, rs_recv_sem, ag_send_sem, ag_recv_sem, copy_sem):
        me = lax.axis_index("i")
        left = lax.rem(me + N_DEV - 1, N_DEV)
        right = lax.rem(me + 1, N_DEV)
        scale = sx_ref[0] * sw_ref[0]

        barrier = pltpu.get_barrier_semaphore()
        for nbr in (left, right):
            pl.semaphore_signal(barrier, inc=1, device_id=(nbr,),
                                device_id_type=pl.DeviceIdType.MESH)
        pl.semaphore_wait(barrier, 2)

        def partial(c):
            xs = x_ref[pl.ds(c * chunk, chunk), :]
            return lax.dot_general(xs, w_ref[...], (((1,), (0,)), ((), ())),
                                   preferred_element_type=jnp.float32)

        for h in range(N_DEV - 1):
            s_chunk = lax.rem(me + N_DEV - h, N_DEV)
            p = partial(s_chunk)
            if h > 0:
                p = p + rs_recv[h - 1].astype(jnp.float32)
            send_buf[h % 2] = p.astype(jnp.bfloat16)
            rdma = pltpu.make_async_remote_copy(
                src_ref=send_buf.at[h % 2],
                dst_ref=rs_recv.at[h],
                send_sem=rs_send_sem.at[h],
                recv_sem=rs_recv_sem.at[h],
                device_id=(right,),
                device_id_type=pl.DeviceIdType.MESH,
            )
            rdma.start()
            rdma.wait()

        own = lax.rem(me + 1, N_DEV)
        y = (partial(own) + rs_recv[N_DEV - 2].astype(jnp.float32)) * scale
        z = y * (1.0 / (1.0 + jnp.exp(-jnp.clip(y, -60.0, 60.0))))
        acc[...] = z
        for c in range(N_DEV):
            @pl.when(own == c)
            def _():
                ag_buf[c] = acc[...].astype(jnp.bfloat16)
        cp = pltpu.make_async_copy(
            acc, out_ref.at[pl.ds(own * chunk, chunk), :], copy_sem)
        cp.start()
        cp.wait()

        for a in range(N_DEV - 1):
            c_send = lax.rem(own + N_DEV - a, N_DEV)
            c_recv = lax.rem(own + N_DEV - a - 1, N_DEV)
            rdma = pltpu.make_async_remote_copy(
                src_ref=ag_buf.at[c_send],
                dst_ref=ag_buf.at[c_send],
                send_sem=ag_send_sem.at[a],
                recv_sem=ag_recv_sem.at[a],
                device_id=(right,),
                device_id_type=pl.DeviceIdType.MESH,
            )
            rdma.start()
            rdma.wait()
            for c in range(N_DEV):
                @pl.when(c_recv == c)
                def _():
                    acc[...] = ag_buf[c].astype(jnp.float32)
            cp = pltpu.make_async_copy(
                acc, out_ref.at[pl.ds(c_recv * chunk, chunk), :], copy_sem)
            cp.start()
            cp.wait()

    return pl.pallas_call(
        body,
        out_shape=jax.ShapeDtypeStruct((m, n), jnp.float32),
        in_specs=[
            pl.BlockSpec(memory_space=pltpu.VMEM),
            pl.BlockSpec(memory_space=pltpu.VMEM),
            pl.BlockSpec(memory_space=pltpu.SMEM),
            pl.BlockSpec(memory_space=pltpu.SMEM),
        ],
        out_specs=pl.BlockSpec(memory_space=pl.ANY),
        scratch_shapes=[
            pltpu.VMEM((chunk, n), jnp.float32),
            pltpu.VMEM((2, chunk, n), jnp.bfloat16),
            pltpu.VMEM((N_DEV - 1, chunk, n), jnp.bfloat16),
            pltpu.VMEM((N_DEV, chunk, n), jnp.bfloat16),
            pltpu.SemaphoreType.DMA((N_DEV - 1,)),
            pltpu.SemaphoreType.DMA((N_DEV - 1,)),
            pltpu.SemaphoreType.DMA((N_DEV - 1,)),
            pltpu.SemaphoreType.DMA((N_DEV - 1,)),
            pltpu.SemaphoreType.DMA,
        ],
        compiler_params=pltpu.CompilerParams(
            collective_id=0, vmem_limit_bytes=100 * 1024 * 1024),
    )(xb, wb, scale_x, scale_w)


# device time: 208039 ns/iter; 1.7354x vs baseline; 1.7354x over previous
import jax
import jax.numpy as jnp
from jax import lax
from jax.experimental import pallas as pl
from jax.experimental.pallas import tpu as pltpu

N_DEV = 4


def kernel(x, w_mat, scale_x, scale_w):
    m, _ = x.shape
    n = w_mat.shape[1]
    chunk = m // N_DEV
    n2 = n // 2

    xb = x.astype(jnp.bfloat16)
    wb = w_mat.astype(jnp.bfloat16)

    def body(x_ref, w_ref, sx_ref, sw_ref, out_ref,
             accs, send_bufs, rs_recv, ag_bufs,
             rs_send_sem, rs_recv_sem, ag_send_sem, ag_recv_sem, copy_sem):
        me = lax.axis_index("i")
        left = lax.rem(me + N_DEV - 1, N_DEV)
        right = lax.rem(me + 1, N_DEV)
        scale = sx_ref[0] * sw_ref[0]

        peer = (right, left)

        barrier = pltpu.get_barrier_semaphore()
        for nbr in (left, right):
            pl.semaphore_signal(barrier, inc=1, device_id=(nbr,),
                                device_id_type=pl.DeviceIdType.MESH)
        pl.semaphore_wait(barrier, 2)

        def rs_chunk(d, h):
            return lax.rem(me + (N_DEV - h if d == 0 else h), N_DEV)

        def own_chunk(d):
            return lax.rem(me + (1 if d == 0 else N_DEV - 1), N_DEV)

        def ag_chunk(d, a, recv):
            step = a + recv
            o = own_chunk(d)
            return lax.rem(o + (N_DEV - step if d == 0 else step), N_DEV)

        def partial(c, d):
            xs = x_ref[pl.ds(c * chunk, chunk), :]
            ws = w_ref[:, pl.ds(d * n2, n2)]
            return lax.dot_general(xs, ws, (((1,), (0,)), ((), ())),
                                   preferred_element_type=jnp.float32)

        deferred = []
        rs_recvs = [[], []]

        for h in range(N_DEV - 1):
            for d in (0, 1):
                accs[d] = partial(rs_chunk(d, h), d)
            for d in (0, 1):
                if h > 0:
                    prev = rs_recvs[d][h - 1]
                    prev.wait_recv()
                    send_bufs[d, h] = (
                        accs[d] + rs_recv[d, h - 1].astype(jnp.float32)
                    ).astype(jnp.bfloat16)
                else:
                    send_bufs[d, h] = accs[d].astype(jnp.bfloat16)
                rdma = pltpu.make_async_remote_copy(
                    src_ref=send_bufs.at[d, h],
                    dst_ref=rs_recv.at[d, h],
                    send_sem=rs_send_sem.at[d, h],
                    recv_sem=rs_recv_sem.at[d, h],
                    device_id=(peer[d],),
                    device_id_type=pl.DeviceIdType.MESH,
                )
                rdma.start()
                deferred.append(rdma)
                rs_recvs[d].append(rdma)

        out_copies = []
        for d in (0, 1):
            o = own_chunk(d)
            p = partial(o, d)
            rs_recvs[d][N_DEV - 2].wait_recv()
            y = (p + rs_recv[d, N_DEV - 2].astype(jnp.float32)) * scale
            accs[d] = y * (1.0 / (1.0 + jnp.exp(-jnp.clip(y, -60.0, 60.0))))
            for c in range(N_DEV):
                @pl.when(o == c)
                def _():
                    ag_bufs[d, c] = accs[d].astype(jnp.bfloat16)
            cp = pltpu.make_async_copy(
                accs.at[d],
                out_ref.at[pl.ds(o * chunk, chunk), pl.ds(d * n2, n2)],
                copy_sem.at[d])
            cp.start()
            out_copies.append(cp)

        for a in range(N_DEV - 1):
            hop = []
            for d in (0, 1):
                c_send = ag_chunk(d, a, 0)
                rdma = pltpu.make_async_remote_copy(
                    src_ref=ag_bufs.at[d, c_send],
                    dst_ref=ag_bufs.at[d, c_send],
                    send_sem=ag_send_sem.at[d, a],
                    recv_sem=ag_recv_sem.at[d, a],
                    device_id=(peer[d],),
                    device_id_type=pl.DeviceIdType.MESH,
                )
                rdma.start()
                deferred.append(rdma)
                hop.append(rdma)
            for d in (0, 1):
                c_recv = ag_chunk(d, a, 1)
                hop[d].wait_recv()
                out_copies[d].wait()
                for c in range(N_DEV):
                    @pl.when(c_recv == c)
                    def _():
                        accs[d] = ag_bufs[d, c].astype(jnp.float32)
                cp = pltpu.make_async_copy(
                    accs.at[d],
                    out_ref.at[pl.ds(c_recv * chunk, chunk),
                               pl.ds(d * n2, n2)],
                    copy_sem.at[d])
                cp.start()
                out_copies[d] = cp

        for cp in out_copies:
            cp.wait()
        for rdma in deferred:
            rdma.wait_send()

    return pl.pallas_call(
        body,
        out_shape=jax.ShapeDtypeStruct((m, n), jnp.float32),
        in_specs=[
            pl.BlockSpec(memory_space=pltpu.VMEM),
            pl.BlockSpec(memory_space=pltpu.VMEM),
            pl.BlockSpec(memory_space=pltpu.SMEM),
            pl.BlockSpec(memory_space=pltpu.SMEM),
        ],
        out_specs=pl.BlockSpec(memory_space=pl.ANY),
        scratch_shapes=[
            pltpu.VMEM((2, chunk, n2), jnp.float32),
            pltpu.VMEM((2, N_DEV - 1, chunk, n2), jnp.bfloat16),
            pltpu.VMEM((2, N_DEV - 1, chunk, n2), jnp.bfloat16),
            pltpu.VMEM((2, N_DEV, chunk, n2), jnp.bfloat16),
            pltpu.SemaphoreType.DMA((2, N_DEV - 1)),
            pltpu.SemaphoreType.DMA((2, N_DEV - 1)),
            pltpu.SemaphoreType.DMA((2, N_DEV - 1)),
            pltpu.SemaphoreType.DMA((2, N_DEV - 1)),
            pltpu.SemaphoreType.DMA((2,)),
        ],
        compiler_params=pltpu.CompilerParams(
            collective_id=0, vmem_limit_bytes=100 * 1024 * 1024),
    )(xb, wb, scale_x, scale_w)


# device time: 191695 ns/iter; 1.8833x vs baseline; 1.0853x over previous
import jax
import jax.numpy as jnp
from jax import lax
from jax.experimental import pallas as pl
from jax.experimental.pallas import tpu as pltpu

N_DEV = 4
S = 2


def kernel(x, w_mat, scale_x, scale_w):
    m, k_loc = x.shape
    n = w_mat.shape[1]
    chunk = m // N_DEV
    n2 = n // 2
    sub = chunk // S

    xq = x.astype(jnp.float8_e4m3fn)
    wq = w_mat.astype(jnp.float8_e5m2)

    def body(x_ref, w_ref, sx_ref, sw_ref, out_ref,
             w_bf, x_stage, accs, send_bufs, rs_recv, ag_bufs,
             rs_send_sem, rs_recv_sem, ag_send_sem, ag_recv_sem, copy_sem):
        me = lax.axis_index("i")
        left = lax.rem(me + N_DEV - 1, N_DEV)
        right = lax.rem(me + 1, N_DEV)
        scale = sx_ref[0] * sw_ref[0]

        peer = (right, left)

        barrier = pltpu.get_barrier_semaphore()
        for nbr in (left, right):
            pl.semaphore_signal(barrier, inc=1, device_id=(nbr,),
                                device_id_type=pl.DeviceIdType.MESH)
        pl.semaphore_wait(barrier, 2)

        w_bf[...] = w_ref[...].astype(jnp.bfloat16)

        def rs_chunk(d, h):
            return lax.rem(me + (N_DEV - h if d == 0 else h), N_DEV)

        def own_chunk(d):
            return lax.rem(me + (1 if d == 0 else N_DEV - 1), N_DEV)

        def ag_chunk(d, a, recv):
            step = a + recv
            o = own_chunk(d)
            return lax.rem(o + (N_DEV - step if d == 0 else step), N_DEV)

        def partial(c, d):
            for cc in range(N_DEV):
                @pl.when(c == cc)
                def _():
                    x_stage[d] = x_ref[pl.ds(cc * chunk, chunk), :].astype(
                        jnp.bfloat16)
            ws = w_bf[:, pl.ds(d * n2, n2)]
            return lax.dot_general(x_stage[d], ws, (((1,), (0,)), ((), ())),
                                   preferred_element_type=jnp.float32)

        def epilogue(y):
            return y * (1.0 / (1.0 + jnp.exp(-jnp.clip(y, -60.0, 60.0))))

        def rs_rdma(d, h, s):
            rows = pl.ds(s * sub, sub)
            return pltpu.make_async_remote_copy(
                src_ref=send_bufs.at[d, h, rows],
                dst_ref=rs_recv.at[d, h, rows],
                send_sem=rs_send_sem.at[d, h, s],
                recv_sem=rs_recv_sem.at[d, h, s],
                device_id=(peer[d],),
                device_id_type=pl.DeviceIdType.MESH,
            )

        def ag_rdma(d, a, s):
            c = ag_chunk(d, a, 0)
            return pltpu.make_async_remote_copy(
                src_ref=ag_bufs.at[d, s, c],
                dst_ref=ag_bufs.at[d, s, c],
                send_sem=ag_send_sem.at[d, a, s],
                recv_sem=ag_recv_sem.at[d, a, s],
                device_id=(peer[d],),
                device_id_type=pl.DeviceIdType.MESH,
            )

        deferred = []
        rs_rdmas = [[None] * S for _ in range(2)]
        ag_rdmas = [[None] * S for _ in range(2)]

        for h in range(N_DEV - 1):
            for d in (0, 1):
                accs[d] = partial(rs_chunk(d, h), d)
            for s in range(S):
                rows = pl.ds(s * sub, sub)
                for d in (0, 1):
                    if h > 0:
                        rs_rdmas[d][s].wait_recv()
                        send_bufs[d, h, rows] = (
                            accs[d, rows]
                            + rs_recv[d, h - 1, rows].astype(jnp.float32)
                        ).astype(jnp.bfloat16)
                    else:
                        send_bufs[d, h, rows] = accs[d, rows].astype(
                            jnp.bfloat16)
                    r = rs_rdma(d, h, s)
                    r.start()
                    deferred.append(r)
                    rs_rdmas[d][s] = r

        for d in (0, 1):
            accs[d] = partial(own_chunk(d), d)
        for s in range(S):
            rows = pl.ds(s * sub, sub)
            for d in (0, 1):
                o = own_chunk(d)
                rs_rdmas[d][s].wait_recv()
                sum_bf = (
                    accs[d, rows]
                    + rs_recv[d, N_DEV - 2, rows].astype(jnp.float32)
                ).astype(jnp.bfloat16)
                for c in range(N_DEV):
                    @pl.when(o == c)
                    def _():
                        ag_bufs[d, s, c] = sum_bf
                r = ag_rdma(d, 0, s)
                r.start()
                deferred.append(r)
                ag_rdmas[d][s] = r

        out_copies = [None, None]
        for d in (0, 1):
            o = own_chunk(d)
            accs[d] = epilogue(
                (accs[d] + rs_recv[d, N_DEV - 2].astype(jnp.float32)) * scale)
            cp = pltpu.make_async_copy(
                accs.at[d],
                out_ref.at[pl.ds(o * chunk, chunk), pl.ds(d * n2, n2)],
                copy_sem.at[d])
            cp.start()
            out_copies[d] = cp

        for a in range(N_DEV - 1):
            for s in range(S):
                for d in (0, 1):
                    ag_rdmas[d][s].wait_recv()
                    if a < N_DEV - 2:
                        r = ag_rdma(d, a + 1, s)
                        r.start()
                        deferred.append(r)
                        ag_rdmas[d][s] = r
            for d in (0, 1):
                c_recv = ag_chunk(d, a, 1)
                out_copies[d].wait()
                for c in range(N_DEV):
                    @pl.when(c_recv == c)
                    def _():
                        for s in range(S):
                            accs[d, pl.ds(s * sub, sub)] = (
                                ag_bufs[d, s, c].astype(jnp.float32))
                accs[d] = epilogue(accs[d] * scale)
                cp = pltpu.make_async_copy(
                    accs.at[d],
                    out_ref.at[pl.ds(c_recv * chunk, chunk),
                               pl.ds(d * n2, n2)],
                    copy_sem.at[d])
                cp.start()
                out_copies[d] = cp

        for cp in out_copies:
            cp.wait()
        for r in deferred:
            r.wait_send()

    return pl.pallas_call(
        body,
        out_shape=jax.ShapeDtypeStruct((m, n), jnp.float32),
        in_specs=[
            pl.BlockSpec(memory_space=pltpu.VMEM),
            pl.BlockSpec(memory_space=pltpu.VMEM),
            pl.BlockSpec(memory_space=pltpu.SMEM),
            pl.BlockSpec(memory_space=pltpu.SMEM),
        ],
        out_specs=pl.BlockSpec(memory_space=pl.ANY),
        scratch_shapes=[
            pltpu.VMEM((k_loc, n), jnp.bfloat16),
            pltpu.VMEM((2, chunk, k_loc), jnp.bfloat16),
            pltpu.VMEM((2, chunk, n2), jnp.float32),
            pltpu.VMEM((2, N_DEV - 1, chunk, n2), jnp.bfloat16),
            pltpu.VMEM((2, N_DEV - 1, chunk, n2), jnp.bfloat16),
            pltpu.VMEM((2, S, N_DEV, sub, n2), jnp.bfloat16),
            pltpu.SemaphoreType.DMA((2, N_DEV - 1, S)),
            pltpu.SemaphoreType.DMA((2, N_DEV - 1, S)),
            pltpu.SemaphoreType.DMA((2, N_DEV - 1, S)),
            pltpu.SemaphoreType.DMA((2, N_DEV - 1, S)),
            pltpu.SemaphoreType.DMA((2,)),
        ],
        compiler_params=pltpu.CompilerParams(
            collective_id=0, vmem_limit_bytes=100 * 1024 * 1024),
    )(xq, wq, scale_x, scale_w)


# device time: 189500 ns/iter; 1.9052x vs baseline; 1.0116x over previous
import jax
import jax.numpy as jnp
from jax import lax
from jax.experimental import pallas as pl
from jax.experimental.pallas import tpu as pltpu

N_DEV = 4
S = 2


def kernel(x, w_mat, scale_x, scale_w):
    m, k_loc = x.shape
    n = w_mat.shape[1]
    chunk = m // N_DEV
    n2 = n // 2
    sub = chunk // S

    xq = x.astype(jnp.float8_e4m3fn)
    wq = w_mat.astype(jnp.float8_e5m2)

    def body(x_ref, w_ref, sx_ref, sw_ref, out_ref,
             w_bf, x_stage, accs, send_bufs, rs_recv, ag_bufs,
             rs_send_sem, rs_recv_sem, ag_send_sem, ag_recv_sem, copy_sem):
        me = lax.axis_index("i")
        left = lax.rem(me + N_DEV - 1, N_DEV)
        right = lax.rem(me + 1, N_DEV)
        scale = sx_ref[0] * sw_ref[0]

        peer = (right, left)

        w_bf[...] = w_ref[...].astype(jnp.bfloat16)

        def rs_chunk(d, h):
            return lax.rem(me + (N_DEV - h if d == 0 else h), N_DEV)

        def own_chunk(d):
            return lax.rem(me + (1 if d == 0 else N_DEV - 1), N_DEV)

        def ag_chunk(d, a, recv):
            step = a + recv
            o = own_chunk(d)
            return lax.rem(o + (N_DEV - step if d == 0 else step), N_DEV)

        def partial(c, d):
            for cc in range(N_DEV):
                @pl.when(c == cc)
                def _():
                    x_stage[d] = x_ref[pl.ds(cc * chunk, chunk), :].astype(
                        jnp.bfloat16)
            ws = w_bf[:, pl.ds(d * n2, n2)]
            return lax.dot_general(x_stage[d], ws, (((1,), (0,)), ((), ())),
                                   preferred_element_type=jnp.float32)

        def epilogue(y):
            return y * (1.0 / (1.0 + jnp.exp(-jnp.clip(y, -60.0, 60.0))))

        def rs_rdma(d, h, s):
            rows = pl.ds(s * sub, sub)
            return pltpu.make_async_remote_copy(
                src_ref=send_bufs.at[d, h, rows],
                dst_ref=rs_recv.at[d, h, rows],
                send_sem=rs_send_sem.at[d, h, s],
                recv_sem=rs_recv_sem.at[d, h, s],
                device_id=(peer[d],),
                device_id_type=pl.DeviceIdType.MESH,
            )

        def ag_rdma(d, a, s):
            c = ag_chunk(d, a, 0)
            return pltpu.make_async_remote_copy(
                src_ref=ag_bufs.at[d, s, c],
                dst_ref=ag_bufs.at[d, s, c],
                send_sem=ag_send_sem.at[d, a, s],
                recv_sem=ag_recv_sem.at[d, a, s],
                device_id=(peer[d],),
                device_id_type=pl.DeviceIdType.MESH,
            )

        deferred = []
        rs_rdmas = [[None] * S for _ in range(2)]
        ag_rdmas = [[None] * S for _ in range(2)]

        for d in (0, 1):
            accs[d] = partial(rs_chunk(d, 0), d)
        for s in range(S):
            rows = pl.ds(s * sub, sub)
            for d in (0, 1):
                send_bufs[d, 0, rows] = accs[d, rows].astype(jnp.bfloat16)

        barrier = pltpu.get_barrier_semaphore()
        for nbr in (left, right):
            pl.semaphore_signal(barrier, inc=1, device_id=(nbr,),
                                device_id_type=pl.DeviceIdType.MESH)
        pl.semaphore_wait(barrier, 2)

        for s in range(S):
            for d in (0, 1):
                r = rs_rdma(d, 0, s)
                r.start()
                deferred.append(r)
                rs_rdmas[d][s] = r

        for h in range(1, N_DEV - 1):
            for d in (0, 1):
                accs[d] = partial(rs_chunk(d, h), d)
            for s in range(S):
                rows = pl.ds(s * sub, sub)
                for d in (0, 1):
                    rs_rdmas[d][s].wait_recv()
                    send_bufs[d, h, rows] = (
                        accs[d, rows]
                        + rs_recv[d, h - 1, rows].astype(jnp.float32)
                    ).astype(jnp.bfloat16)
                    r = rs_rdma(d, h, s)
                    r.start()
                    deferred.append(r)
                    rs_rdmas[d][s] = r

        for d in (0, 1):
            accs[d] = partial(own_chunk(d), d)
        for s in range(S):
            rows = pl.ds(s * sub, sub)
            for d in (0, 1):
                o = own_chunk(d)
                rs_rdmas[d][s].wait_recv()
                sum_bf = (
                    accs[d, rows]
                    + rs_recv[d, N_DEV - 2, rows].astype(jnp.float32)
                ).astype(jnp.bfloat16)
                for c in range(N_DEV):
                    @pl.when(o == c)
                    def _():
                        ag_bufs[d, s, c] = sum_bf
                r = ag_rdma(d, 0, s)
                r.start()
                deferred.append(r)
                ag_rdmas[d][s] = r

        out_copies = [[None] * S for _ in range(2)]
        for d in (0, 1):
            o = own_chunk(d)
            accs[d] = epilogue(
                (accs[d] + rs_recv[d, N_DEV - 2].astype(jnp.float32)) * scale)
            for s in range(S):
                rows = pl.ds(s * sub, sub)
                cp = pltpu.make_async_copy(
                    accs.at[d, rows],
                    out_ref.at[pl.ds(o * chunk + s * sub, sub),
                               pl.ds(d * n2, n2)],
                    copy_sem.at[d, s])
                cp.start()
                out_copies[d][s] = cp

        for a in range(N_DEV - 1):
            for s in range(S):
                rows = pl.ds(s * sub, sub)
                for d in (0, 1):
                    ag_rdmas[d][s].wait_recv()
                    if a < N_DEV - 2:
                        r = ag_rdma(d, a + 1, s)
                        r.start()
                        deferred.append(r)
                        ag_rdmas[d][s] = r
                    c_recv = ag_chunk(d, a, 1)
                    out_copies[d][s].wait()
                    for c in range(N_DEV):
                        @pl.when(c_recv == c)
                        def _():
                            accs[d, rows] = epilogue(
                                ag_bufs[d, s, c].astype(jnp.float32) * scale)
                    cp = pltpu.make_async_copy(
                        accs.at[d, rows],
                        out_ref.at[pl.ds(c_recv * chunk + s * sub, sub),
                                   pl.ds(d * n2, n2)],
                        copy_sem.at[d, s])
                    cp.start()
                    out_copies[d][s] = cp

        for cps in out_copies:
            for cp in cps:
                cp.wait()
        for r in deferred:
            r.wait_send()

    return pl.pallas_call(
        body,
        out_shape=jax.ShapeDtypeStruct((m, n), jnp.float32),
        in_specs=[
            pl.BlockSpec(memory_space=pltpu.VMEM),
            pl.BlockSpec(memory_space=pltpu.VMEM),
            pl.BlockSpec(memory_space=pltpu.SMEM),
            pl.BlockSpec(memory_space=pltpu.SMEM),
        ],
        out_specs=pl.BlockSpec(memory_space=pl.ANY),
        scratch_shapes=[
            pltpu.VMEM((k_loc, n), jnp.bfloat16),
            pltpu.VMEM((2, chunk, k_loc), jnp.bfloat16),
            pltpu.VMEM((2, chunk, n2), jnp.float32),
            pltpu.VMEM((2, N_DEV - 1, chunk, n2), jnp.bfloat16),
            pltpu.VMEM((2, N_DEV - 1, chunk, n2), jnp.bfloat16),
            pltpu.VMEM((2, S, N_DEV, sub, n2), jnp.bfloat16),
            pltpu.SemaphoreType.DMA((2, N_DEV - 1, S)),
            pltpu.SemaphoreType.DMA((2, N_DEV - 1, S)),
            pltpu.SemaphoreType.DMA((2, N_DEV - 1, S)),
            pltpu.SemaphoreType.DMA((2, N_DEV - 1, S)),
            pltpu.SemaphoreType.DMA((2, S)),
        ],
        compiler_params=pltpu.CompilerParams(
            collective_id=0, vmem_limit_bytes=100 * 1024 * 1024),
    )(xq, wq, scale_x, scale_w)


# device time: 189082 ns/iter; 1.9094x vs baseline; 1.0022x over previous
import jax
import jax.numpy as jnp
from jax import lax
from jax.experimental import pallas as pl
from jax.experimental.pallas import tpu as pltpu

N_DEV = 4
S = 4


def kernel(x, w_mat, scale_x, scale_w):
    m, k_loc = x.shape
    n = w_mat.shape[1]
    chunk = m // N_DEV
    n2 = n // 2
    sub = chunk // S

    xq = x.astype(jnp.float8_e4m3fn)
    wq = w_mat.astype(jnp.float8_e5m2)

    def body(x_ref, w_ref, sx_ref, sw_ref, out_ref,
             w_bf, x_stage, accs, send_bufs, rs_recv, ag_bufs,
             rs_send_sem, rs_recv_sem, ag_send_sem, ag_recv_sem, copy_sem):
        me = lax.axis_index("i")
        left = lax.rem(me + N_DEV - 1, N_DEV)
        right = lax.rem(me + 1, N_DEV)
        scale = sx_ref[0] * sw_ref[0]

        peer = (right, left)

        w_bf[...] = w_ref[...].astype(jnp.bfloat16)

        def rs_chunk(d, h):
            return lax.rem(me + (N_DEV - h if d == 0 else h), N_DEV)

        def own_chunk(d):
            return lax.rem(me + (1 if d == 0 else N_DEV - 1), N_DEV)

        def ag_chunk(d, a, recv):
            step = a + recv
            o = own_chunk(d)
            return lax.rem(o + (N_DEV - step if d == 0 else step), N_DEV)

        def partial(c, d):
            for cc in range(N_DEV):
                @pl.when(c == cc)
                def _():
                    x_stage[d] = x_ref[pl.ds(cc * chunk, chunk), :].astype(
                        jnp.bfloat16)
            ws = w_bf[:, pl.ds(d * n2, n2)]
            return lax.dot_general(x_stage[d], ws, (((1,), (0,)), ((), ())),
                                   preferred_element_type=jnp.float32)

        def epilogue(y):
            return y * (1.0 / (1.0 + jnp.exp(-jnp.clip(y, -60.0, 60.0))))

        def rs_rdma(d, h, s):
            rows = pl.ds(s * sub, sub)
            return pltpu.make_async_remote_copy(
                src_ref=send_bufs.at[d, h, rows],
                dst_ref=rs_recv.at[d, h, rows],
                send_sem=rs_send_sem.at[d, h, s],
                recv_sem=rs_recv_sem.at[d, h, s],
                device_id=(peer[d],),
                device_id_type=pl.DeviceIdType.MESH,
            )

        def ag_rdma(d, a, s):
            c = ag_chunk(d, a, 0)
            return pltpu.make_async_remote_copy(
                src_ref=ag_bufs.at[d, s, c],
                dst_ref=ag_bufs.at[d, s, c],
                send_sem=ag_send_sem.at[d, a, s],
                recv_sem=ag_recv_sem.at[d, a, s],
                device_id=(peer[d],),
                device_id_type=pl.DeviceIdType.MESH,
            )

        deferred = []
        rs_rdmas = [[None] * S for _ in range(2)]
        ag_rdmas = [[None] * S for _ in range(2)]

        for d in (0, 1):
            accs[d] = partial(rs_chunk(d, 0), d)
        for s in range(S):
            rows = pl.ds(s * sub, sub)
            for d in (0, 1):
                send_bufs[d, 0, rows] = accs[d, rows].astype(jnp.bfloat16)

        barrier = pltpu.get_barrier_semaphore()
        for nbr in (left, right):
            pl.semaphore_signal(barrier, inc=1, device_id=(nbr,),
                                device_id_type=pl.DeviceIdType.MESH)
        pl.semaphore_wait(barrier, 2)

        for s in range(S):
            for d in (0, 1):
                r = rs_rdma(d, 0, s)
                r.start()
                deferred.append(r)
                rs_rdmas[d][s] = r

        for h in range(1, N_DEV - 1):
            for d in (0, 1):
                accs[d] = partial(rs_chunk(d, h), d)
            for s in range(S):
                rows = pl.ds(s * sub, sub)
                for d in (0, 1):
                    rs_rdmas[d][s].wait_recv()
                    send_bufs[d, h, rows] = (
                        accs[d, rows]
                        + rs_recv[d, h - 1, rows].astype(jnp.float32)
                    ).astype(jnp.bfloat16)
                    r = rs_rdma(d, h, s)
                    r.start()
                    deferred.append(r)
                    rs_rdmas[d][s] = r

        for d in (0, 1):
            accs[d] = partial(own_chunk(d), d)
        for s in range(S):
            rows = pl.ds(s * sub, sub)
            for d in (0, 1):
                o = own_chunk(d)
                rs_rdmas[d][s].wait_recv()
                sum_bf = (
                    accs[d, rows]
                    + rs_recv[d, N_DEV - 2, rows].astype(jnp.float32)
                ).astype(jnp.bfloat16)
                for c in range(N_DEV):
                    @pl.when(o == c)
                    def _():
                        ag_bufs[d, s, c] = sum_bf
                r = ag_rdma(d, 0, s)
                r.start()
                deferred.append(r)
                ag_rdmas[d][s] = r

        out_copies = [[None] * S for _ in range(2)]
        for d in (0, 1):
            o = own_chunk(d)
            accs[d] = epilogue(
                (accs[d] + rs_recv[d, N_DEV - 2].astype(jnp.float32)) * scale)
            for s in range(S):
                rows = pl.ds(s * sub, sub)
                cp = pltpu.make_async_copy(
                    accs.at[d, rows],
                    out_ref.at[pl.ds(o * chunk + s * sub, sub),
                               pl.ds(d * n2, n2)],
                    copy_sem.at[d, s])
                cp.start()
                out_copies[d][s] = cp

        for a in range(N_DEV - 1):
            for s in range(S):
                rows = pl.ds(s * sub, sub)
                for d in (0, 1):
                    ag_rdmas[d][s].wait_recv()
                    if a < N_DEV - 2:
                        r = ag_rdma(d, a + 1, s)
                        r.start()
                        deferred.append(r)
                        ag_rdmas[d][s] = r
                    c_recv = ag_chunk(d, a, 1)
                    out_copies[d][s].wait()
                    for c in range(N_DEV):
                        @pl.when(c_recv == c)
                        def _():
                            accs[d, rows] = epilogue(
                                ag_bufs[d, s, c].astype(jnp.float32) * scale)
                    cp = pltpu.make_async_copy(
                        accs.at[d, rows],
                        out_ref.at[pl.ds(c_recv * chunk + s * sub, sub),
                                   pl.ds(d * n2, n2)],
                        copy_sem.at[d, s])
                    cp.start()
                    out_copies[d][s] = cp

        for cps in out_copies:
            for cp in cps:
                cp.wait()
        for r in deferred:
            r.wait_send()

    return pl.pallas_call(
        body,
        out_shape=jax.ShapeDtypeStruct((m, n), jnp.float32),
        in_specs=[
            pl.BlockSpec(memory_space=pltpu.VMEM),
            pl.BlockSpec(memory_space=pltpu.VMEM),
            pl.BlockSpec(memory_space=pltpu.SMEM),
            pl.BlockSpec(memory_space=pltpu.SMEM),
        ],
        out_specs=pl.BlockSpec(memory_space=pl.ANY),
        scratch_shapes=[
            pltpu.VMEM((k_loc, n), jnp.bfloat16),
            pltpu.VMEM((2, chunk, k_loc), jnp.bfloat16),
            pltpu.VMEM((2, chunk, n2), jnp.float32),
            pltpu.VMEM((2, N_DEV - 1, chunk, n2), jnp.bfloat16),
            pltpu.VMEM((2, N_DEV - 1, chunk, n2), jnp.bfloat16),
            pltpu.VMEM((2, S, N_DEV, sub, n2), jnp.bfloat16),
            pltpu.SemaphoreType.DMA((2, N_DEV - 1, S)),
            pltpu.SemaphoreType.DMA((2, N_DEV - 1, S)),
            pltpu.SemaphoreType.DMA((2, N_DEV - 1, S)),
            pltpu.SemaphoreType.DMA((2, N_DEV - 1, S)),
            pltpu.SemaphoreType.DMA((2, S)),
        ],
        compiler_params=pltpu.CompilerParams(
            collective_id=0, vmem_limit_bytes=100 * 1024 * 1024),
    )(xq, wq, scale_x, scale_w)


# device time: 175983 ns/iter; 2.0515x vs baseline; 1.0744x over previous
import jax
import jax.numpy as jnp
from jax import lax
from jax.experimental import pallas as pl
from jax.experimental.pallas import tpu as pltpu

N_DEV = 4
S = 2
N_SLOT = 2


def kernel(x, w_mat, scale_x, scale_w):
    m, k_loc = x.shape
    n = w_mat.shape[1]
    chunk = m // N_DEV
    n2 = n // 2
    sub = chunk // S

    def body(x_ref, w_ref, sx_ref, sw_ref, out_ref,
             w_bf, x_stage, x32, accs, send_bufs, rs_recv, ag_bufs,
             in_sem, rs_send_sem, rs_recv_sem, ag_send_sem, ag_recv_sem,
             copy_sem):
        me = lax.axis_index("i")
        left = lax.rem(me + N_DEV - 1, N_DEV)
        right = lax.rem(me + 1, N_DEV)
        scale = sx_ref[0] * sw_ref[0]

        peer = (right, left)

        def rs_chunk(d, h):
            return lax.rem(me + (N_DEV - h if d == 0 else h), N_DEV)

        def own_chunk(d):
            return lax.rem(me + (1 if d == 0 else N_DEV - 1), N_DEV)

        def ag_chunk(d, a, recv):
            step = a + recv
            o = own_chunk(d)
            return lax.rem(o + (N_DEV - step if d == 0 else step), N_DEV)

        def start_x_dma(d, c):
            cp = pltpu.make_async_copy(
                x_ref.at[pl.ds(c * chunk, chunk), :], x32.at[d],
                in_sem.at[d])
            cp.start()
            return cp

        def dot(d):
            accs[d] = lax.dot_general(
                x_stage[d], w_bf[:, pl.ds(d * n2, n2)],
                (((1,), (0,)), ((), ())),
                preferred_element_type=jnp.float32)

        def epilogue(y):
            return y * (1.0 / (1.0 + jnp.exp(-jnp.clip(y, -60.0, 60.0))))

        def rs_rdma(d, h, s):
            rows = pl.ds(s * sub, sub)
            return pltpu.make_async_remote_copy(
                src_ref=send_bufs.at[d, h % N_SLOT, rows],
                dst_ref=rs_recv.at[d, h, rows],
                send_sem=rs_send_sem.at[d, h % N_SLOT, s],
                recv_sem=rs_recv_sem.at[d, h, s],
                device_id=(peer[d],),
                device_id_type=pl.DeviceIdType.MESH,
            )

        def ag_rdma(d, a, s):
            c = ag_chunk(d, a, 0)
            return pltpu.make_async_remote_copy(
                src_ref=ag_bufs.at[d, s, c],
                dst_ref=ag_bufs.at[d, s, c],
                send_sem=ag_send_sem.at[d, a, s],
                recv_sem=ag_recv_sem.at[d, a, s],
                device_id=(peer[d],),
                device_id_type=pl.DeviceIdType.MESH,
            )

        unsent = []
        rs_rdmas = [[None] * S for _ in range(2)]
        slot_rdmas = [[[None] * S for _ in range(N_SLOT)]
                      for _ in range(2)]
        ag_rdmas = [[None] * S for _ in range(2)]

        w_dmas = []
        for d in (0, 1):
            cp = pltpu.make_async_copy(
                w_ref.at[:, pl.ds(d * n2, n2)], accs.at[d], copy_sem.at[d, 0])
            cp.start()
            w_dmas.append(cp)
        x_dmas = [start_x_dma(d, rs_chunk(d, 0)) for d in (0, 1)]
        for d in (0, 1):
            w_dmas[d].wait()
            w_bf[:, pl.ds(d * n2, n2)] = accs[d].astype(jnp.bfloat16)
        for d in (0, 1):
            x_dmas[d].wait()
            x_stage[d] = x32[d].astype(jnp.bfloat16)
            dot(d)
        for s in range(S):
            rows = pl.ds(s * sub, sub)
            for d in (0, 1):
                send_bufs[d, 0, rows] = accs[d, rows].astype(jnp.bfloat16)
        x_dmas = [start_x_dma(d, rs_chunk(d, 1)) for d in (0, 1)]

        barrier = pltpu.get_barrier_semaphore()
        for nbr in (left, right):
            pl.semaphore_signal(barrier, inc=1, device_id=(nbr,),
                                device_id_type=pl.DeviceIdType.MESH)
        pl.semaphore_wait(barrier, 2)

        for s in range(S):
            for d in (0, 1):
                r = rs_rdma(d, 0, s)
                r.start()
                unsent.append(r)
                rs_rdmas[d][s] = r
                slot_rdmas[d][0][s] = r

        for h in range(1, N_DEV - 1):
            for d in (0, 1):
                x_dmas[d].wait()
                x_stage[d] = x32[d].astype(jnp.bfloat16)
                dot(d)
            nxt = rs_chunk if h < N_DEV - 2 else (lambda d, _h: own_chunk(d))
            x_dmas = [start_x_dma(d, nxt(d, h + 1)) for d in (0, 1)]
            for s in range(S):
                rows = pl.ds(s * sub, sub)
                for d in (0, 1):
                    prev = slot_rdmas[d][h % N_SLOT][s]
                    if prev is not None:
                        prev.wait_send()
                        unsent.remove(prev)
                    rs_rdmas[d][s].wait_recv()
                    send_bufs[d, h % N_SLOT, rows] = (
                        accs[d, rows]
                        + rs_recv[d, h - 1, rows].astype(jnp.float32)
                    ).astype(jnp.bfloat16)
                    r = rs_rdma(d, h, s)
                    r.start()
                    unsent.append(r)
                    rs_rdmas[d][s] = r
                    slot_rdmas[d][h % N_SLOT][s] = r

        for d in (0, 1):
            x_dmas[d].wait()
            x_stage[d] = x32[d].astype(jnp.bfloat16)
            dot(d)
        for s in range(S):
            rows = pl.ds(s * sub, sub)
            for d in (0, 1):
                o = own_chunk(d)
                rs_rdmas[d][s].wait_recv()
                sum_bf = (
                    accs[d, rows]
                    + rs_recv[d, N_DEV - 2, rows].astype(jnp.float32)
                ).astype(jnp.bfloat16)
                for c in range(N_DEV):
                    @pl.when(o == c)
                    def _():
                        ag_bufs[d, s, c] = sum_bf
                r = ag_rdma(d, 0, s)
                r.start()
                unsent.append(r)
                ag_rdmas[d][s] = r

        out_copies = [[None] * S for _ in range(2)]
        for d in (0, 1):
            o = own_chunk(d)
            accs[d] = epilogue(
                (accs[d] + rs_recv[d, N_DEV - 2].astype(jnp.float32)) * scale)
            for s in range(S):
                rows = pl.ds(s * sub, sub)
                cp = pltpu.make_async_copy(
                    accs.at[d, rows],
                    out_ref.at[pl.ds(o * chunk + s * sub, sub),
                               pl.ds(d * n2, n2)],
                    copy_sem.at[d, s])
                cp.start()
                out_copies[d][s] = cp

        for a in range(N_DEV - 1):
            for s in range(S):
                rows = pl.ds(s * sub, sub)
                for d in (0, 1):
                    ag_rdmas[d][s].wait_recv()
                    if a < N_DEV - 2:
                        r = ag_rdma(d, a + 1, s)
                        r.start()
                        unsent.append(r)
                        ag_rdmas[d][s] = r
                    c_recv = ag_chunk(d, a, 1)
                    out_copies[d][s].wait()
                    for c in range(N_DEV):
                        @pl.when(c_recv == c)
                        def _():
                            accs[d, rows] = epilogue(
                                ag_bufs[d, s, c].astype(jnp.float32) * scale)
                    cp = pltpu.make_async_copy(
                        accs.at[d, rows],
                        out_ref.at[pl.ds(c_recv * chunk + s * sub, sub),
                                   pl.ds(d * n2, n2)],
                        copy_sem.at[d, s])
                    cp.start()
                    out_copies[d][s] = cp

        for cps in out_copies:
            for cp in cps:
                cp.wait()
        for r in unsent:
            r.wait_send()

    return pl.pallas_call(
        body,
        out_shape=jax.ShapeDtypeStruct((m, n), jnp.float32),
        in_specs=[
            pl.BlockSpec(memory_space=pl.ANY),
            pl.BlockSpec(memory_space=pl.ANY),
            pl.BlockSpec(memory_space=pltpu.SMEM),
            pl.BlockSpec(memory_space=pltpu.SMEM),
        ],
        out_specs=pl.BlockSpec(memory_space=pl.ANY),
        scratch_shapes=[
            pltpu.VMEM((k_loc, n), jnp.bfloat16),
            pltpu.VMEM((2, chunk, k_loc), jnp.bfloat16),
            pltpu.VMEM((2, chunk, k_loc), jnp.float32),
            pltpu.VMEM((2, chunk, n2), jnp.float32),
            pltpu.VMEM((2, N_SLOT, chunk, n2), jnp.bfloat16),
            pltpu.VMEM((2, N_DEV - 1, chunk, n2), jnp.bfloat16),
            pltpu.VMEM((2, S, N_DEV, sub, n2), jnp.bfloat16),
            pltpu.SemaphoreType.DMA((2,)),
            pltpu.SemaphoreType.DMA((2, N_SLOT, S)),
            pltpu.SemaphoreType.DMA((2, N_DEV - 1, S)),
            pltpu.SemaphoreType.DMA((2, N_DEV - 1, S)),
            pltpu.SemaphoreType.DMA((2, N_DEV - 1, S)),
            pltpu.SemaphoreType.DMA((2, S)),
        ],
        compiler_params=pltpu.CompilerParams(
            collective_id=0, vmem_limit_bytes=100 * 1024 * 1024),
    )(x, w_mat, scale_x, scale_w)


# device time: 174354 ns/iter; 2.0707x vs baseline; 1.0093x over previous
import jax
import jax.numpy as jnp
from jax import lax
from jax.experimental import pallas as pl
from jax.experimental.pallas import tpu as pltpu

N_DEV = 4
S = 2
N_SLOT = 2


def kernel(x, w_mat, scale_x, scale_w):
    m, k_loc = x.shape
    n = w_mat.shape[1]
    chunk = m // N_DEV
    n2 = n // 2
    sub = chunk // S

    def body(x_ref, w_ref, sx_ref, sw_ref, out_ref,
             w_bf, x_stage, x32, accs, send_bufs, rs_recv, ag_bufs,
             in_sem, rs_send_sem, rs_recv_sem, ag_send_sem, ag_recv_sem,
             copy_sem):
        me = lax.axis_index("i")
        left = lax.rem(me + N_DEV - 1, N_DEV)
        right = lax.rem(me + 1, N_DEV)
        scale = sx_ref[0] * sw_ref[0]

        peer = (right, left)

        def rs_chunk(d, h):
            return lax.rem(me + (N_DEV - h if d == 0 else h), N_DEV)

        def own_chunk(d):
            return lax.rem(me + (1 if d == 0 else N_DEV - 1), N_DEV)

        def ag_chunk(d, a, recv):
            step = a + recv
            o = own_chunk(d)
            return lax.rem(o + (N_DEV - step if d == 0 else step), N_DEV)

        def start_x_dma(d, c):
            cp = pltpu.make_async_copy(
                x_ref.at[pl.ds(c * chunk, chunk), :], x32.at[d],
                in_sem.at[d])
            cp.start()
            return cp

        def dot(d):
            accs[d] = lax.dot_general(
                x_stage[d], w_bf[:, pl.ds(d * n2, n2)],
                (((1,), (0,)), ((), ())),
                preferred_element_type=jnp.float32)

        def epilogue(y):
            return y * (1.0 / (1.0 + jnp.exp(-jnp.clip(y, -60.0, 60.0))))

        def rs_rdma(d, h, s):
            rows = pl.ds(s * sub, sub)
            return pltpu.make_async_remote_copy(
                src_ref=send_bufs.at[d, h % N_SLOT, rows],
                dst_ref=rs_recv.at[d, h, rows],
                send_sem=rs_send_sem.at[d, h % N_SLOT, s],
                recv_sem=rs_recv_sem.at[d, h, s],
                device_id=(peer[d],),
                device_id_type=pl.DeviceIdType.MESH,
            )

        def ag_rdma(d, a, s):
            c = ag_chunk(d, a, 0)
            return pltpu.make_async_remote_copy(
                src_ref=ag_bufs.at[d, s, c],
                dst_ref=ag_bufs.at[d, s, c],
                send_sem=ag_send_sem.at[d, a, s],
                recv_sem=ag_recv_sem.at[d, a, s],
                device_id=(peer[d],),
                device_id_type=pl.DeviceIdType.MESH,
            )

        unsent = []
        rs_rdmas = [[None] * S for _ in range(2)]
        slot_rdmas = [[[None] * S for _ in range(N_SLOT)]
                      for _ in range(2)]
        ag_rdmas = [[None] * S for _ in range(2)]

        w_dmas = []
        for d in (0, 1):
            cp = pltpu.make_async_copy(
                w_ref.at[:, pl.ds(d * n2, n2)], accs.at[d], copy_sem.at[d, 0])
            cp.start()
            w_dmas.append(cp)
        x_dmas = [start_x_dma(d, rs_chunk(d, 0)) for d in (0, 1)]
        for d in (0, 1):
            w_dmas[d].wait()
            w_bf[:, pl.ds(d * n2, n2)] = accs[d].astype(jnp.bfloat16)
        rows0 = pl.ds(0, sub)
        for d in (0, 1):
            x_dmas[d].wait()
            x_stage[d] = x32[d].astype(jnp.bfloat16)
            accs[d, rows0] = lax.dot_general(
                x_stage[d, rows0], w_bf[:, pl.ds(d * n2, n2)],
                (((1,), (0,)), ((), ())),
                preferred_element_type=jnp.float32)
            send_bufs[d, 0, rows0] = accs[d, rows0].astype(jnp.bfloat16)

        barrier = pltpu.get_barrier_semaphore()
        for nbr in (left, right):
            pl.semaphore_signal(barrier, inc=1, device_id=(nbr,),
                                device_id_type=pl.DeviceIdType.MESH)
        pl.semaphore_wait(barrier, 2)

        for d in (0, 1):
            r = rs_rdma(d, 0, 0)
            r.start()
            unsent.append(r)
            rs_rdmas[d][0] = r
            slot_rdmas[d][0][0] = r
        for s in range(1, S):
            rows = pl.ds(s * sub, sub)
            for d in (0, 1):
                accs[d, rows] = lax.dot_general(
                    x_stage[d, rows], w_bf[:, pl.ds(d * n2, n2)],
                    (((1,), (0,)), ((), ())),
                    preferred_element_type=jnp.float32)
                send_bufs[d, 0, rows] = accs[d, rows].astype(jnp.bfloat16)
                r = rs_rdma(d, 0, s)
                r.start()
                unsent.append(r)
                rs_rdmas[d][s] = r
                slot_rdmas[d][0][s] = r
        x_dmas = [start_x_dma(d, rs_chunk(d, 1)) for d in (0, 1)]

        for h in range(1, N_DEV - 1):
            for d in (0, 1):
                x_dmas[d].wait()
                x_stage[d] = x32[d].astype(jnp.bfloat16)
                dot(d)
            nxt = rs_chunk if h < N_DEV - 2 else (lambda d, _h: own_chunk(d))
            x_dmas = [start_x_dma(d, nxt(d, h + 1)) for d in (0, 1)]
            for s in range(S):
                rows = pl.ds(s * sub, sub)
                for d in (0, 1):
                    prev = slot_rdmas[d][h % N_SLOT][s]
                    if prev is not None:
                        prev.wait_send()
                        unsent.remove(prev)
                    rs_rdmas[d][s].wait_recv()
                    send_bufs[d, h % N_SLOT, rows] = (
                        accs[d, rows]
                        + rs_recv[d, h - 1, rows].astype(jnp.float32)
                    ).astype(jnp.bfloat16)
                    r = rs_rdma(d, h, s)
                    r.start()
                    unsent.append(r)
                    rs_rdmas[d][s] = r
                    slot_rdmas[d][h % N_SLOT][s] = r

        for d in (0, 1):
            x_dmas[d].wait()
            x_stage[d] = x32[d].astype(jnp.bfloat16)
            dot(d)
        for s in range(S):
            rows = pl.ds(s * sub, sub)
            for d in (0, 1):
                o = own_chunk(d)
                rs_rdmas[d][s].wait_recv()
                sum_bf = (
                    accs[d, rows]
                    + rs_recv[d, N_DEV - 2, rows].astype(jnp.float32)
                ).astype(jnp.bfloat16)
                for c in range(N_DEV):
                    @pl.when(o == c)
                    def _():
                        ag_bufs[d, s, c] = sum_bf
                r = ag_rdma(d, 0, s)
                r.start()
                unsent.append(r)
                ag_rdmas[d][s] = r

        out_copies = [[None] * S for _ in range(2)]
        for d in (0, 1):
            o = own_chunk(d)
            accs[d] = epilogue(
                (accs[d] + rs_recv[d, N_DEV - 2].astype(jnp.float32)) * scale)
            for s in range(S):
                rows = pl.ds(s * sub, sub)
                cp = pltpu.make_async_copy(
                    accs.at[d, rows],
                    out_ref.at[pl.ds(o * chunk + s * sub, sub),
                               pl.ds(d * n2, n2)],
                    copy_sem.at[d, s])
                cp.start()
                out_copies[d][s] = cp

        for a in range(N_DEV - 1):
            for s in range(S):
                rows = pl.ds(s * sub, sub)
                for d in (0, 1):
                    ag_rdmas[d][s].wait_recv()
                    if a < N_DEV - 2:
                        r = ag_rdma(d, a + 1, s)
                        r.start()
                        unsent.append(r)
                        ag_rdmas[d][s] = r
                    c_recv = ag_chunk(d, a, 1)
                    out_copies[d][s].wait()
                    for c in range(N_DEV):
                        @pl.when(c_recv == c)
                        def _():
                            accs[d, rows] = epilogue(
                                ag_bufs[d, s, c].astype(jnp.float32) * scale)
                    cp = pltpu.make_async_copy(
                        accs.at[d, rows],
                        out_ref.at[pl.ds(c_recv * chunk + s * sub, sub),
                                   pl.ds(d * n2, n2)],
                        copy_sem.at[d, s])
                    cp.start()
                    out_copies[d][s] = cp

        for cps in out_copies:
            for cp in cps:
                cp.wait()
        for r in unsent:
            r.wait_send()

    return pl.pallas_call(
        body,
        out_shape=jax.ShapeDtypeStruct((m, n), jnp.float32),
        in_specs=[
            pl.BlockSpec(memory_space=pl.ANY),
            pl.BlockSpec(memory_space=pl.ANY),
            pl.BlockSpec(memory_space=pltpu.SMEM),
            pl.BlockSpec(memory_space=pltpu.SMEM),
        ],
        out_specs=pl.BlockSpec(memory_space=pl.ANY),
        scratch_shapes=[
            pltpu.VMEM((k_loc, n), jnp.bfloat16),
            pltpu.VMEM((2, chunk, k_loc), jnp.bfloat16),
            pltpu.VMEM((2, chunk, k_loc), jnp.float32),
            pltpu.VMEM((2, chunk, n2), jnp.float32),
            pltpu.VMEM((2, N_SLOT, chunk, n2), jnp.bfloat16),
            pltpu.VMEM((2, N_DEV - 1, chunk, n2), jnp.bfloat16),
            pltpu.VMEM((2, S, N_DEV, sub, n2), jnp.bfloat16),
            pltpu.SemaphoreType.DMA((2,)),
            pltpu.SemaphoreType.DMA((2, N_SLOT, S)),
            pltpu.SemaphoreType.DMA((2, N_DEV - 1, S)),
            pltpu.SemaphoreType.DMA((2, N_DEV - 1, S)),
            pltpu.SemaphoreType.DMA((2, N_DEV - 1, S)),
            pltpu.SemaphoreType.DMA((2, S)),
        ],
        compiler_params=pltpu.CompilerParams(
            collective_id=0, vmem_limit_bytes=100 * 1024 * 1024),
    )(x, w_mat, scale_x, scale_w)
